# Initial kernel scaffold; baseline (speedup 1.0000x reference)
import jax
import jax.numpy as jnp
from jax.experimental import pallas as pl


def kernel(x, edge_index, W1, b1, W2, b2):
    raise NotImplementedError("write your pallas kernel here")


# speedup vs baseline: 6.2222x; 6.2222x over previous
import functools

import jax
import jax.numpy as jnp
from jax import lax
from jax.experimental import pallas as pl
from jax.experimental.pallas import tpu as pltpu
from jax.experimental.pallas import tpu_sc as plsc

N = 10000
E = 160000
D_IN = 256
D_HID = 512
D_OUT = 256

NC = 2
NS = 16
CH = 128
OPW = 16
E_PAD = 163840
CHUNKS = E_PAD // NS // CH
DEG_CHUNKS = E_PAD // (NC * NS) // CH
N_PAD = 10240
ROWS_PER_TILE = N_PAD // NS
N2 = NC * N_PAD

_MESH = plsc.VectorSubcoreMesh(core_axis_name="c", subcore_axis_name="s")


def _sc_deg_body(dstp, ones_hbm, zeros_hbm, deg_out, idx_v, ones_v, stage_v,
                 acc_sh, sem):
    c = lax.axis_index("c")
    s = lax.axis_index("s")
    off = c * N_PAD

    pltpu.sync_copy(ones_hbm, ones_v)
    pltpu.sync_copy(zeros_hbm, stage_v)
    for i in range(ROWS_PER_TILE // CH):
        pltpu.sync_copy(stage_v, acc_sh.at[pl.ds(s * ROWS_PER_TILE + i * CH, CH)])
    plsc.subcore_barrier()

    base = (c * NS + s) * (DEG_CHUNKS * CH)

    def chunk(j, carry):
        pltpu.sync_copy(dstp.at[pl.ds(base + j * CH, CH)], idx_v)
        for k in range(CH // OPW):
            didx = idx_v[pl.ds(k * OPW, OPW)]
            pltpu.sync_copy(ones_v, acc_sh.at[didx], add=True)
        return carry

    lax.fori_loop(0, DEG_CHUNKS, chunk, 0)
    plsc.subcore_barrier()
    for i in range(ROWS_PER_TILE // CH):
        r = s * ROWS_PER_TILE + i * CH
        pltpu.sync_copy(acc_sh.at[pl.ds(r, CH)], stage_v)
        pltpu.sync_copy(stage_v, deg_out.at[pl.ds(off + r, CH)])


_sc_deg = pl.kernel(
    _sc_deg_body,
    out_type=jax.ShapeDtypeStruct((N2, 128), jnp.float32),
    mesh=_MESH,
    scratch_types=[
        pltpu.VMEM((CH,), jnp.int32),
        pltpu.VMEM((OPW, 128), jnp.float32),
        pltpu.VMEM((CH, 128), jnp.float32),
        pltpu.VMEM_SHARED((N_PAD, 128), jnp.float32),
        pltpu.SemaphoreType.DMA,
    ],
)


def _sc_agg_body(feat, srcp, dstp, zeros_hbm, out, src_v, dst_v, rows_v,
                 stage_v, acc_sh, sem):
    c = lax.axis_index("c")
    s = lax.axis_index("s")
    off = c * N_PAD

    pltpu.sync_copy(zeros_hbm, stage_v)
    for i in range(ROWS_PER_TILE // CH):
        pltpu.sync_copy(stage_v, acc_sh.at[pl.ds(s * ROWS_PER_TILE + i * CH, CH)])
    plsc.subcore_barrier()

    base = s * (CHUNKS * CH)

    def chunk(j, carry):
        pltpu.sync_copy(srcp.at[pl.ds(base + j * CH, CH)], src_v)
        pltpu.sync_copy(dstp.at[pl.ds(base + j * CH, CH)], dst_v)
        copies = []
        for k in range(CH // OPW):
            sidx = src_v[pl.ds(k * OPW, OPW)] + off
            copies.append(pltpu.async_copy(
                feat.at[sidx], rows_v.at[pl.ds(k * OPW, OPW)], sem))
        for cp in copies:
            cp.wait()
        for k in range(CH // OPW):
            didx = dst_v[pl.ds(k * OPW, OPW)]
            pltpu.sync_copy(rows_v.at[pl.ds(k * OPW, OPW)], acc_sh.at[didx],
                            add=True)
        return carry

    lax.fori_loop(0, CHUNKS, chunk, 0)
    plsc.subcore_barrier()
    for i in range(ROWS_PER_TILE // CH):
        r = s * ROWS_PER_TILE + i * CH
        pltpu.sync_copy(acc_sh.at[pl.ds(r, CH)], stage_v)
        pltpu.sync_copy(stage_v, out.at[pl.ds(off + r, CH)])


_sc_agg = pl.kernel(
    _sc_agg_body,
    out_type=jax.ShapeDtypeStruct((N2, 128), jnp.float32),
    mesh=_MESH,
    scratch_types=[
        pltpu.VMEM((CH,), jnp.int32),
        pltpu.VMEM((CH,), jnp.int32),
        pltpu.VMEM((CH, 128), jnp.float32),
        pltpu.VMEM((CH, 128), jnp.float32),
        pltpu.VMEM_SHARED((N_PAD, 128), jnp.float32),
        pltpu.SemaphoreType.DMA,
    ],
)


_BA = 256


def _row_mask(i, b):
    rows = i * b + lax.broadcasted_iota(jnp.int32, (b, 1), 0)
    return rows < N


def _tc_pre_body(x_ref, deg_ref, xs_ref, degsum_ref):
    mask = _row_mask(pl.program_id(0), _BA)
    deg = deg_ref[0] + deg_ref[1]
    dinv = lax.rsqrt(deg[:, 0:1] + 1.0)
    xs = jnp.where(mask, x_ref[...] * dinv, 0.0)
    xs_ref[0] = xs[:, :128]
    xs_ref[1] = xs[:, 128:]
    degsum_ref[...] = deg[:, :16]


def _tc_mid_body(s1_ref, x_ref, deg_ref, w1_ref, b1_ref, w2_ref, b2_ref,
                 ys_ref, t_ref):
    mask = _row_mask(pl.program_id(0), _BA)
    dinv = lax.rsqrt(deg_ref[:, 0:1] + 1.0)
    x_blk = jnp.where(mask, x_ref[...], 0.0)
    s1 = jnp.concatenate([s1_ref[0], s1_ref[1]], axis=1)
    agg1 = dinv * s1 + (dinv * dinv) * x_blk
    h1 = jnp.maximum(
        jnp.dot(agg1, w1_ref[...], preferred_element_type=jnp.float32)
        + b1_ref[...], 0.0)
    y = jnp.dot(h1, w2_ref[...], preferred_element_type=jnp.float32)
    ys = jnp.where(mask, y * dinv, 0.0)
    ys_ref[0] = ys[:, :128]
    ys_ref[1] = ys[:, 128:]
    t_ref[...] = (dinv * dinv) * y + b2_ref[...]


def _tc_post_body(s2_ref, t_ref, deg_ref, o_ref):
    dinv = lax.rsqrt(deg_ref[:, 0:1] + 1.0)
    s2 = jnp.concatenate([s2_ref[0], s2_ref[1]], axis=1)
    o_ref[...] = dinv * s2 + t_ref[...]


def _rows(i):
    return (i, 0)


def _halves(i):
    return (0, i, 0)


def _whole(i):
    return (0, 0)


_tc_pre = pl.pallas_call(
    _tc_pre_body,
    grid=(N_PAD // _BA,),
    in_specs=[
        pl.BlockSpec((_BA, D_IN), _rows),
        pl.BlockSpec((NC, _BA, 128), _halves),
    ],
    out_specs=[
        pl.BlockSpec((NC, _BA, 128), _halves),
        pl.BlockSpec((_BA, 16), _rows),
    ],
    out_shape=[
        jax.ShapeDtypeStruct((NC, N_PAD, 128), jnp.float32),
        jax.ShapeDtypeStruct((N_PAD, 16), jnp.float32),
    ],
)

_tc_mid = pl.pallas_call(
    _tc_mid_body,
    grid=(N_PAD // _BA,),
    in_specs=[
        pl.BlockSpec((NC, _BA, 128), _halves),
        pl.BlockSpec((_BA, D_IN), _rows),
        pl.BlockSpec((_BA, 16), _rows),
        pl.BlockSpec((D_IN, D_HID), _whole),
        pl.BlockSpec((1, D_HID), _whole),
        pl.BlockSpec((D_HID, D_OUT), _whole),
        pl.BlockSpec((1, D_OUT), _whole),
    ],
    out_specs=[
        pl.BlockSpec((NC, _BA, 128), _halves),
        pl.BlockSpec((_BA, D_OUT), _rows),
    ],
    out_shape=[
        jax.ShapeDtypeStruct((NC, N_PAD, 128), jnp.float32),
        jax.ShapeDtypeStruct((N_PAD, D_OUT), jnp.float32),
    ],
)

_tc_post = pl.pallas_call(
    _tc_post_body,
    grid=(N_PAD // _BA,),
    in_specs=[
        pl.BlockSpec((NC, _BA, 128), _halves),
        pl.BlockSpec((_BA, D_OUT), _rows),
        pl.BlockSpec((_BA, 16), _rows),
    ],
    out_specs=pl.BlockSpec((_BA, D_OUT), _rows),
    out_shape=jax.ShapeDtypeStruct((N, D_OUT), jnp.float32),
)


def kernel(x, edge_index, W1, b1, W2, b2):
    src = edge_index[0].astype(jnp.int32)
    dst = edge_index[1].astype(jnp.int32)
    pad = jnp.full((E_PAD - E,), N, jnp.int32)
    srcp = jnp.concatenate([src, pad])
    dstp = jnp.concatenate([dst, pad])
    ones128 = jnp.ones((OPW, 128), jnp.float32)
    zeros128 = jnp.zeros((CH, 128), jnp.float32)

    deg2 = _sc_deg(dstp, ones128, zeros128)
    xs, deg16 = _tc_pre(x, deg2.reshape(NC, N_PAD, 128))
    s1 = _sc_agg(xs.reshape(N2, 128), srcp, dstp, zeros128)
    ys, t = _tc_mid(s1.reshape(NC, N_PAD, 128), x, deg16, W1,
                    b1.reshape(1, -1), W2, b2.reshape(1, -1))
    s2 = _sc_agg(ys.reshape(N2, 128), srcp, dstp, zeros128)
    return _tc_post(s2.reshape(NC, N_PAD, 128), t, deg16)


# speedup vs baseline: 15.5263x; 2.4953x over previous
import functools

import jax
import jax.numpy as jnp
from jax import lax
from jax.experimental import pallas as pl
from jax.experimental.pallas import tpu as pltpu
from jax.experimental.pallas import tpu_sc as plsc

N = 10000
E = 160000
D_IN = 256
D_HID = 512
D_OUT = 256

NC = 2
NS = 16
CH = 128
OPW = 16
E_PAD = 163840
CHUNKS = E_PAD // NS // CH
DEG_CHUNKS = E_PAD // (NC * NS) // CH
N_PAD = 10240
ROWS_PER_TILE = N_PAD // NS
N2 = NC * N_PAD

_MESH = plsc.VectorSubcoreMesh(core_axis_name="c", subcore_axis_name="s")


def _sc_deg_body(dstp, ones_hbm, zeros_hbm, deg_out, idx_v, ones_v, stage_v,
                 acc_sh, sem):
    c = lax.axis_index("c")
    s = lax.axis_index("s")
    off = c * N_PAD

    pltpu.sync_copy(ones_hbm, ones_v)
    pltpu.sync_copy(zeros_hbm, stage_v)
    for i in range(ROWS_PER_TILE // CH):
        pltpu.sync_copy(stage_v, acc_sh.at[pl.ds(s * ROWS_PER_TILE + i * CH, CH)])
    plsc.subcore_barrier()

    base = (c * NS + s) * (DEG_CHUNKS * CH)

    def chunk(j, carry):
        pltpu.sync_copy(dstp.at[pl.ds(base + j * CH, CH)], idx_v)
        for k in range(CH // OPW):
            didx = idx_v[pl.ds(k * OPW, OPW)]
            pltpu.sync_copy(ones_v, acc_sh.at[didx], add=True)
        return carry

    lax.fori_loop(0, DEG_CHUNKS, chunk, 0)
    plsc.subcore_barrier()
    for i in range(ROWS_PER_TILE // CH):
        r = s * ROWS_PER_TILE + i * CH
        pltpu.sync_copy(acc_sh.at[pl.ds(r, CH)], stage_v)
        pltpu.sync_copy(stage_v, deg_out.at[pl.ds(off + r, CH)])


_sc_deg = pl.kernel(
    _sc_deg_body,
    out_type=jax.ShapeDtypeStruct((N2, 128), jnp.float32),
    mesh=_MESH,
    scratch_types=[
        pltpu.VMEM((CH,), jnp.int32),
        pltpu.VMEM((OPW, 128), jnp.float32),
        pltpu.VMEM((CH, 128), jnp.float32),
        pltpu.VMEM_SHARED((N_PAD, 128), jnp.float32),
        pltpu.SemaphoreType.DMA,
    ],
)


def _sc_agg_body(feat, srcp, dstp, zeros_hbm, out, src_v, dst_v, rows_v,
                 stage_v, acc_sh, gsem, ssem, isem):
    c = lax.axis_index("c")
    s = lax.axis_index("s")
    off = c * N_PAD

    for i in range(ROWS_PER_TILE // CH):
        pltpu.sync_copy(zeros_hbm,
                        acc_sh.at[pl.ds(s * ROWS_PER_TILE + i * CH, CH)])
    plsc.subcore_barrier()

    base = s * (CHUNKS * CH)

    def fire_gathers(j, b):
        for k in range(CH // OPW):
            sidx = src_v[pl.ds(b * CH + k * OPW, OPW)] + off
            pltpu.async_copy(feat.at[sidx],
                             rows_v.at[pl.ds(b * CH + k * OPW, OPW)], gsem)

    pltpu.sync_copy(srcp.at[pl.ds(base, CH)], src_v.at[pl.ds(0, CH)])
    pltpu.sync_copy(dstp.at[pl.ds(base, CH)], dst_v.at[pl.ds(0, CH)])
    fire_gathers(0, 0)
    pltpu.async_copy(srcp.at[pl.ds(base + CH, CH)], src_v.at[pl.ds(CH, CH)], isem)
    pltpu.async_copy(dstp.at[pl.ds(base + CH, CH)], dst_v.at[pl.ds(CH, CH)], isem)

    def chunk(j, carry):
        b = j % 2
        bn = (j + 1) % 2
        pltpu.make_async_copy(zeros_hbm, rows_v.at[pl.ds(0, CH)], gsem).wait()
        for k in range(CH // OPW):
            didx = dst_v[pl.ds(b * CH + k * OPW, OPW)]
            pltpu.async_copy(rows_v.at[pl.ds(b * CH + k * OPW, OPW)],
                             acc_sh.at[didx], ssem, add=True)
        @pl.when(j + 2 < CHUNKS)
        def _():
            pltpu.async_copy(srcp.at[pl.ds(base + (j + 2) * CH, CH)],
                             src_v.at[pl.ds(b * CH, CH)], isem)
            pltpu.async_copy(dstp.at[pl.ds(base + (j + 2) * CH, CH)],
                             dst_v.at[pl.ds(b * CH, CH)], isem)
        @pl.when(j + 1 < CHUNKS)
        def _():
            @pl.when(j > 0)
            def _():
                pltpu.make_async_copy(zeros_hbm, rows_v.at[pl.ds(0, CH)],
                                      ssem).wait()
            pltpu.make_async_copy(srcp.at[pl.ds(0, CH)],
                                  src_v.at[pl.ds(0, CH)], isem).wait()
            pltpu.make_async_copy(dstp.at[pl.ds(0, CH)],
                                  dst_v.at[pl.ds(0, CH)], isem).wait()
            fire_gathers(j + 1, bn)
        return carry

    lax.fori_loop(0, CHUNKS, chunk, 0)
    pltpu.make_async_copy(zeros_hbm, rows_v.at[pl.ds(0, CH)], ssem).wait()
    pltpu.make_async_copy(zeros_hbm, rows_v.at[pl.ds(0, CH)], ssem).wait()
    plsc.subcore_barrier()
    for i in range(ROWS_PER_TILE // 64):
        r = s * ROWS_PER_TILE + i * 64
        pltpu.sync_copy(acc_sh.at[pl.ds(r, 64)], stage_v)
        pltpu.sync_copy(stage_v, out.at[pl.ds(off + r, 64)])


_sc_agg = pl.kernel(
    _sc_agg_body,
    out_type=jax.ShapeDtypeStruct((N2, 128), jnp.float32),
    mesh=_MESH,
    scratch_types=[
        pltpu.VMEM((2 * CH,), jnp.int32),
        pltpu.VMEM((2 * CH,), jnp.int32),
        pltpu.VMEM((2 * CH, 128), jnp.float32),
        pltpu.VMEM((64, 128), jnp.float32),
        pltpu.VMEM_SHARED((N_PAD, 128), jnp.float32),
        pltpu.SemaphoreType.DMA,
        pltpu.SemaphoreType.DMA,
        pltpu.SemaphoreType.DMA,
    ],
)


_BA = 256


def _row_mask(i, b):
    rows = i * b + lax.broadcasted_iota(jnp.int32, (b, 1), 0)
    return rows < N


def _tc_pre_body(x_ref, deg_ref, xs_ref, degsum_ref):
    mask = _row_mask(pl.program_id(0), _BA)
    deg = deg_ref[0] + deg_ref[1]
    dinv = lax.rsqrt(deg[:, 0:1] + 1.0)
    xs = jnp.where(mask, x_ref[...] * dinv, 0.0)
    xs_ref[0] = xs[:, :128]
    xs_ref[1] = xs[:, 128:]
    degsum_ref[...] = deg[:, :16]


def _tc_mid_body(s1_ref, x_ref, deg_ref, w1_ref, b1_ref, w2_ref, b2_ref,
                 ys_ref, t_ref):
    mask = _row_mask(pl.program_id(0), _BA)
    dinv = lax.rsqrt(deg_ref[:, 0:1] + 1.0)
    x_blk = jnp.where(mask, x_ref[...], 0.0)
    s1 = jnp.concatenate([s1_ref[0], s1_ref[1]], axis=1)
    agg1 = dinv * s1 + (dinv * dinv) * x_blk
    h1 = jnp.maximum(
        jnp.dot(agg1, w1_ref[...], preferred_element_type=jnp.float32)
        + b1_ref[...], 0.0)
    y = jnp.dot(h1, w2_ref[...], preferred_element_type=jnp.float32)
    ys = jnp.where(mask, y * dinv, 0.0)
    ys_ref[0] = ys[:, :128]
    ys_ref[1] = ys[:, 128:]
    t_ref[...] = (dinv * dinv) * y + b2_ref[...]


def _tc_post_body(s2_ref, t_ref, deg_ref, o_ref):
    dinv = lax.rsqrt(deg_ref[:, 0:1] + 1.0)
    s2 = jnp.concatenate([s2_ref[0], s2_ref[1]], axis=1)
    o_ref[...] = dinv * s2 + t_ref[...]


def _rows(i):
    return (i, 0)


def _halves(i):
    return (0, i, 0)


def _whole(i):
    return (0, 0)


_tc_pre = pl.pallas_call(
    _tc_pre_body,
    grid=(N_PAD // _BA,),
    in_specs=[
        pl.BlockSpec((_BA, D_IN), _rows),
        pl.BlockSpec((NC, _BA, 128), _halves),
    ],
    out_specs=[
        pl.BlockSpec((NC, _BA, 128), _halves),
        pl.BlockSpec((_BA, 16), _rows),
    ],
    out_shape=[
        jax.ShapeDtypeStruct((NC, N_PAD, 128), jnp.float32),
        jax.ShapeDtypeStruct((N_PAD, 16), jnp.float32),
    ],
)

_tc_mid = pl.pallas_call(
    _tc_mid_body,
    grid=(N_PAD // _BA,),
    in_specs=[
        pl.BlockSpec((NC, _BA, 128), _halves),
        pl.BlockSpec((_BA, D_IN), _rows),
        pl.BlockSpec((_BA, 16), _rows),
        pl.BlockSpec((D_IN, D_HID), _whole),
        pl.BlockSpec((1, D_HID), _whole),
        pl.BlockSpec((D_HID, D_OUT), _whole),
        pl.BlockSpec((1, D_OUT), _whole),
    ],
    out_specs=[
        pl.BlockSpec((NC, _BA, 128), _halves),
        pl.BlockSpec((_BA, D_OUT), _rows),
    ],
    out_shape=[
        jax.ShapeDtypeStruct((NC, N_PAD, 128), jnp.float32),
        jax.ShapeDtypeStruct((N_PAD, D_OUT), jnp.float32),
    ],
)

_tc_post = pl.pallas_call(
    _tc_post_body,
    grid=(N_PAD // _BA,),
    in_specs=[
        pl.BlockSpec((NC, _BA, 128), _halves),
        pl.BlockSpec((_BA, D_OUT), _rows),
        pl.BlockSpec((_BA, 16), _rows),
    ],
    out_specs=pl.BlockSpec((_BA, D_OUT), _rows),
    out_shape=jax.ShapeDtypeStruct((N, D_OUT), jnp.float32),
)


def kernel(x, edge_index, W1, b1, W2, b2):
    src = edge_index[0].astype(jnp.int32)
    dst = edge_index[1].astype(jnp.int32)
    pad = N + jnp.arange(E_PAD - E, dtype=jnp.int32) % (N_PAD - N)
    srcp = jnp.concatenate([src, pad])
    dstp = jnp.concatenate([dst, pad])
    ones128 = jnp.ones((OPW, 128), jnp.float32)
    zeros128 = jnp.zeros((CH, 128), jnp.float32)

    deg2 = _sc_deg(dstp, ones128, zeros128)
    xs, deg16 = _tc_pre(x, deg2.reshape(NC, N_PAD, 128))
    s1 = _sc_agg(xs.reshape(N2, 128), srcp, dstp, zeros128)
    ys, t = _tc_mid(s1.reshape(NC, N_PAD, 128), x, deg16, W1,
                    b1.reshape(1, -1), W2, b2.reshape(1, -1))
    s2 = _sc_agg(ys.reshape(N2, 128), srcp, dstp, zeros128)
    return _tc_post(s2.reshape(NC, N_PAD, 128), t, deg16)


# speedup vs baseline: 18.7686x; 1.2088x over previous
import functools

import jax
import jax.numpy as jnp
from jax import lax
from jax.experimental import pallas as pl
from jax.experimental.pallas import tpu as pltpu
from jax.experimental.pallas import tpu_sc as plsc

N = 10000
E = 160000
D_IN = 256
D_HID = 512
D_OUT = 256

NC = 2
NS = 16
CH = 128
OPW = 16
E_PAD = 163840
CHUNKS = E_PAD // NS // CH
DEG_CHUNKS = E_PAD // (NC * NS) // CH
N_PAD = 10240
ROWS_PER_TILE = N_PAD // NS
N2 = NC * N_PAD

_MESH = plsc.VectorSubcoreMesh(core_axis_name="c", subcore_axis_name="s")


def _sc_deg_body(dstp, ones_hbm, zeros_hbm, deg_out, idx_v, ones_v, stage_v,
                 acc_sh, ssem, isem):
    c = lax.axis_index("c")
    s = lax.axis_index("s")
    off = c * N_PAD

    pltpu.sync_copy(ones_hbm, ones_v)
    pltpu.sync_copy(zeros_hbm, stage_v)
    for i in range(ROWS_PER_TILE // CH):
        pltpu.sync_copy(stage_v, acc_sh.at[pl.ds(s * ROWS_PER_TILE + i * CH, CH)])
    plsc.subcore_barrier()

    base = (c * NS + s) * (DEG_CHUNKS * CH)

    pltpu.sync_copy(dstp.at[pl.ds(base, CH)], idx_v.at[pl.ds(0, CH)])

    def chunk(j, carry):
        b = j % 2
        bn = (j + 1) % 2
        @pl.when(j > 0)
        def _():
            pltpu.make_async_copy(zeros_hbm, stage_v, ssem).wait()
            pltpu.make_async_copy(dstp.at[pl.ds(0, CH)],
                                  idx_v.at[pl.ds(0, CH)], isem).wait()
        for k in range(CH // OPW):
            didx = idx_v[pl.ds(b * CH + k * OPW, OPW)]
            pltpu.async_copy(ones_v, acc_sh.at[didx], ssem, add=True)
        @pl.when(j + 1 < DEG_CHUNKS)
        def _():
            pltpu.async_copy(dstp.at[pl.ds(base + (j + 1) * CH, CH)],
                             idx_v.at[pl.ds(bn * CH, CH)], isem)
        return carry

    lax.fori_loop(0, DEG_CHUNKS, chunk, 0)
    pltpu.make_async_copy(zeros_hbm, stage_v, ssem).wait()
    plsc.subcore_barrier()
    for i in range(ROWS_PER_TILE // CH):
        r = s * ROWS_PER_TILE + i * CH
        pltpu.sync_copy(acc_sh.at[pl.ds(r, CH)], stage_v)
        pltpu.sync_copy(stage_v, deg_out.at[pl.ds(off + r, CH)])


_sc_deg = pl.kernel(
    _sc_deg_body,
    out_type=jax.ShapeDtypeStruct((N2, 128), jnp.float32),
    mesh=_MESH,
    scratch_types=[
        pltpu.VMEM((2 * CH,), jnp.int32),
        pltpu.VMEM((OPW, 128), jnp.float32),
        pltpu.VMEM((CH, 128), jnp.float32),
        pltpu.VMEM_SHARED((N_PAD, 128), jnp.float32),
        pltpu.SemaphoreType.DMA,
        pltpu.SemaphoreType.DMA,
    ],
)


def _sc_agg_body(feat, srcp, dstp, zeros_hbm, out, src_v, dst_v, rows_v,
                 stage_v, acc_sh, gsem, ssem, isem):
    c = lax.axis_index("c")
    s = lax.axis_index("s")
    off = c * N_PAD

    for i in range(ROWS_PER_TILE // CH):
        pltpu.sync_copy(zeros_hbm,
                        acc_sh.at[pl.ds(s * ROWS_PER_TILE + i * CH, CH)])
    plsc.subcore_barrier()

    base = s * (CHUNKS * CH)

    def fire_gathers(j, b):
        for k in range(CH // OPW):
            sidx = src_v[pl.ds(b * CH + k * OPW, OPW)] + off
            pltpu.async_copy(feat.at[sidx],
                             rows_v.at[pl.ds(b * CH + k * OPW, OPW)], gsem)

    pltpu.sync_copy(srcp.at[pl.ds(base, CH)], src_v.at[pl.ds(0, CH)])
    pltpu.sync_copy(dstp.at[pl.ds(base, CH)], dst_v.at[pl.ds(0, CH)])
    fire_gathers(0, 0)
    pltpu.async_copy(srcp.at[pl.ds(base + CH, CH)], src_v.at[pl.ds(CH, CH)], isem)
    pltpu.async_copy(dstp.at[pl.ds(base + CH, CH)], dst_v.at[pl.ds(CH, CH)], isem)

    def chunk(j, carry):
        b = j % 2
        bn = (j + 1) % 2
        @pl.when(j > 0)
        def _():
            pltpu.make_async_copy(zeros_hbm, rows_v.at[pl.ds(0, CH)],
                                  ssem).wait()
        @pl.when(j + 1 < CHUNKS)
        def _():
            pltpu.make_async_copy(srcp.at[pl.ds(0, CH)],
                                  src_v.at[pl.ds(0, CH)], isem).wait()
            pltpu.make_async_copy(dstp.at[pl.ds(0, CH)],
                                  dst_v.at[pl.ds(0, CH)], isem).wait()
            fire_gathers(j + 1, bn)
        pltpu.make_async_copy(zeros_hbm, rows_v.at[pl.ds(0, CH)], gsem).wait()
        for k in range(CH // OPW):
            didx = dst_v[pl.ds(b * CH + k * OPW, OPW)]
            pltpu.async_copy(rows_v.at[pl.ds(b * CH + k * OPW, OPW)],
                             acc_sh.at[didx], ssem, add=True)
        @pl.when(j + 2 < CHUNKS)
        def _():
            pltpu.async_copy(srcp.at[pl.ds(base + (j + 2) * CH, CH)],
                             src_v.at[pl.ds(b * CH, CH)], isem)
            pltpu.async_copy(dstp.at[pl.ds(base + (j + 2) * CH, CH)],
                             dst_v.at[pl.ds(b * CH, CH)], isem)
        return carry

    lax.fori_loop(0, CHUNKS, chunk, 0)
    pltpu.make_async_copy(zeros_hbm, rows_v.at[pl.ds(0, CH)], ssem).wait()
    plsc.subcore_barrier()
    for i in range(ROWS_PER_TILE // 64):
        r = s * ROWS_PER_TILE + i * 64
        pltpu.sync_copy(acc_sh.at[pl.ds(r, 64)], stage_v)
        pltpu.sync_copy(stage_v, out.at[pl.ds(off + r, 64)])


_sc_agg = pl.kernel(
    _sc_agg_body,
    out_type=jax.ShapeDtypeStruct((N2, 128), jnp.float32),
    mesh=_MESH,
    scratch_types=[
        pltpu.VMEM((2 * CH,), jnp.int32),
        pltpu.VMEM((2 * CH,), jnp.int32),
        pltpu.VMEM((2 * CH, 128), jnp.float32),
        pltpu.VMEM((64, 128), jnp.float32),
        pltpu.VMEM_SHARED((N_PAD, 128), jnp.float32),
        pltpu.SemaphoreType.DMA,
        pltpu.SemaphoreType.DMA,
        pltpu.SemaphoreType.DMA,
    ],
)


_BA = 256


def _row_mask(i, b):
    rows = i * b + lax.broadcasted_iota(jnp.int32, (b, 1), 0)
    return rows < N


def _tc_pre_body(x_ref, deg_ref, xs_ref, degsum_ref):
    mask = _row_mask(pl.program_id(0), _BA)
    deg = deg_ref[0] + deg_ref[1]
    dinv = lax.rsqrt(deg[:, 0:1] + 1.0)
    xs = jnp.where(mask, x_ref[...] * dinv, 0.0)
    xs_ref[0] = xs[:, :128]
    xs_ref[1] = xs[:, 128:]
    degsum_ref[...] = deg[:, :16]


def _tc_mid_body(s1_ref, x_ref, deg_ref, w1_ref, b1_ref, w2_ref, b2_ref,
                 ys_ref, t_ref):
    mask = _row_mask(pl.program_id(0), _BA)
    dinv = lax.rsqrt(deg_ref[:, 0:1] + 1.0)
    x_blk = jnp.where(mask, x_ref[...], 0.0)
    s1 = jnp.concatenate([s1_ref[0], s1_ref[1]], axis=1)
    agg1 = dinv * s1 + (dinv * dinv) * x_blk
    h1 = jnp.maximum(
        jnp.dot(agg1, w1_ref[...], preferred_element_type=jnp.float32)
        + b1_ref[...], 0.0)
    y = jnp.dot(h1, w2_ref[...], preferred_element_type=jnp.float32)
    ys = jnp.where(mask, y * dinv, 0.0)
    ys_ref[0] = ys[:, :128]
    ys_ref[1] = ys[:, 128:]
    t_ref[...] = (dinv * dinv) * y + b2_ref[...]


def _tc_post_body(s2_ref, t_ref, deg_ref, o_ref):
    dinv = lax.rsqrt(deg_ref[:, 0:1] + 1.0)
    s2 = jnp.concatenate([s2_ref[0], s2_ref[1]], axis=1)
    o_ref[...] = dinv * s2 + t_ref[...]


def _rows(i):
    return (i, 0)


def _halves(i):
    return (0, i, 0)


def _whole(i):
    return (0, 0)


_tc_pre = pl.pallas_call(
    _tc_pre_body,
    grid=(N_PAD // _BA,),
    in_specs=[
        pl.BlockSpec((_BA, D_IN), _rows),
        pl.BlockSpec((NC, _BA, 128), _halves),
    ],
    out_specs=[
        pl.BlockSpec((NC, _BA, 128), _halves),
        pl.BlockSpec((_BA, 16), _rows),
    ],
    out_shape=[
        jax.ShapeDtypeStruct((NC, N_PAD, 128), jnp.float32),
        jax.ShapeDtypeStruct((N_PAD, 16), jnp.float32),
    ],
)

_tc_mid = pl.pallas_call(
    _tc_mid_body,
    grid=(N_PAD // _BA,),
    in_specs=[
        pl.BlockSpec((NC, _BA, 128), _halves),
        pl.BlockSpec((_BA, D_IN), _rows),
        pl.BlockSpec((_BA, 16), _rows),
        pl.BlockSpec((D_IN, D_HID), _whole),
        pl.BlockSpec((1, D_HID), _whole),
        pl.BlockSpec((D_HID, D_OUT), _whole),
        pl.BlockSpec((1, D_OUT), _whole),
    ],
    out_specs=[
        pl.BlockSpec((NC, _BA, 128), _halves),
        pl.BlockSpec((_BA, D_OUT), _rows),
    ],
    out_shape=[
        jax.ShapeDtypeStruct((NC, N_PAD, 128), jnp.float32),
        jax.ShapeDtypeStruct((N_PAD, D_OUT), jnp.float32),
    ],
)

_tc_post = pl.pallas_call(
    _tc_post_body,
    grid=(N_PAD // _BA,),
    in_specs=[
        pl.BlockSpec((NC, _BA, 128), _halves),
        pl.BlockSpec((_BA, D_OUT), _rows),
        pl.BlockSpec((_BA, 16), _rows),
    ],
    out_specs=pl.BlockSpec((_BA, D_OUT), _rows),
    out_shape=jax.ShapeDtypeStruct((N, D_OUT), jnp.float32),
)


def kernel(x, edge_index, W1, b1, W2, b2):
    src = edge_index[0].astype(jnp.int32)
    dst = edge_index[1].astype(jnp.int32)
    pad = N + jnp.arange(E_PAD - E, dtype=jnp.int32) % (N_PAD - N)
    srcp = jnp.concatenate([src, pad])
    dstp = jnp.concatenate([dst, pad])
    ones128 = jnp.ones((OPW, 128), jnp.float32)
    zeros128 = jnp.zeros((CH, 128), jnp.float32)

    deg2 = _sc_deg(dstp, ones128, zeros128)
    xs, deg16 = _tc_pre(x, deg2.reshape(NC, N_PAD, 128))
    s1 = _sc_agg(xs.reshape(N2, 128), srcp, dstp, zeros128)
    ys, t = _tc_mid(s1.reshape(NC, N_PAD, 128), x, deg16, W1,
                    b1.reshape(1, -1), W2, b2.reshape(1, -1))
    s2 = _sc_agg(ys.reshape(N2, 128), srcp, dstp, zeros128)
    return _tc_post(s2.reshape(NC, N_PAD, 128), t, deg16)
